# baseline (speedup 1.0000x reference)
import functools

import jax
import jax.numpy as jnp
from jax import lax
from jax.experimental import pallas as pl
from jax.experimental.pallas import tpu as pltpu
from jax.experimental.pallas import tpu_sc as plsc

_NUM_CODES = 8192
_EMBED = 256
_N = 16 * 1024

_TM = 1024
_TN = 1024
_W1 = 2736
_W2 = 5472
_BIG = 2 ** 30
_INF = float("inf")


def _bf16_round(x):
    return x.astype(jnp.bfloat16).astype(jnp.float32)


def _blk_minarg(d, jj, mask=None):
    if mask is None:
        lmin = jnp.min(d, axis=1)
        lidx = jnp.min(jnp.where(d == lmin[:, None], jj, _BIG), axis=1)
    else:
        dm = jnp.where(mask, d, _INF)
        lmin = jnp.min(dm, axis=1)
        lidx = jnp.min(jnp.where(mask & (d == lmin[:, None]), jj, _BIG),
                       axis=1)
    return lmin, lidx


def _merge(vref, iref, lmin, lidx, first):
    if first:
        vref[...] = lmin
        iref[...] = lidx
    else:
        upd = lmin < vref[...]
        vref[...] = jnp.where(upd, lmin, vref[...])
        iref[...] = jnp.where(upd, lidx, iref[...])


def _argmin_body(zn_ref, cn_ref, z_ref, cb_ref, codes_ref, loss_ref,
                 s0v, s0i, s1v, s1i, s2v, s2i, acc):
    r = pl.program_id(0)
    c = pl.program_id(1)
    num_r = pl.num_programs(0)
    num_c = pl.num_programs(1)

    m = lax.dot_general(z_ref[...].astype(jnp.bfloat16),
                        cb_ref[...].astype(jnp.bfloat16),
                        (((1,), (1,)), ((), ())),
                        preferred_element_type=jnp.float32)
    t = zn_ref[...][:, None] + cn_ref[...][None, :]
    d = t - 2.0 * m
    jj = lax.broadcasted_iota(jnp.int32, (_TM, _TN), 1) + c * _TN

    @pl.when(c == 0)
    def _():
        lmin, lidx = _blk_minarg(d, jj)
        _merge(s0v, s0i, lmin, lidx, True)

    @pl.when(c == 1)
    def _():
        lmin, lidx = _blk_minarg(d, jj)
        _merge(s0v, s0i, lmin, lidx, False)

    @pl.when(c == 2)
    def _():
        lminA, lidxA = _blk_minarg(d, jj, jj < _W1)
        _merge(s0v, s0i, lminA, lidxA, False)
        lminB, lidxB = _blk_minarg(d, jj, jj >= _W1)
        _merge(s1v, s1i, lminB, lidxB, True)

    @pl.when((c == 3) | (c == 4))
    def _():
        lmin, lidx = _blk_minarg(d, jj)
        _merge(s1v, s1i, lmin, lidx, False)

    @pl.when(c == 5)
    def _():
        lminA, lidxA = _blk_minarg(d, jj, jj < _W2)
        _merge(s1v, s1i, lminA, lidxA, False)
        lminB, lidxB = _blk_minarg(d, jj, jj >= _W2)
        _merge(s2v, s2i, lminB, lidxB, True)

    @pl.when(c == 6)
    def _():
        lmin, lidx = _blk_minarg(d, jj)
        _merge(s2v, s2i, lmin, lidx, False)

    @pl.when(c == num_c - 1)
    def _():
        lmin, lidx = _blk_minarg(d, jj)
        _merge(s2v, s2i, lmin, lidx, False)

        S = _bf16_round(s0v[...])
        P = s0i[...]
        V = s0v[...]
        u1 = s1v[...] < S
        P = jnp.where(u1, s1i[...], P)
        V = jnp.where(u1, s1v[...], V)
        S = jnp.where(u1, _bf16_round(s1v[...]), S)
        u2 = s2v[...] < S
        P = jnp.where(u2, s2i[...], P)
        V = jnp.where(u2, s2v[...], V)
        codes_ref[...] = P

        s = jnp.sum(V)
        prev = jnp.where(r == 0, jnp.float32(0.0), acc[0])
        tot = prev + s
        acc[0] = tot

        @pl.when(r == num_r - 1)
        def _():
            loss_ref[0, 0] = tot * jnp.float32(1.25 / (_N * _EMBED))


def _argmin_call(zn, cn, z_flat, codebook, interpret=False):
    grid = (_N // _TM, _NUM_CODES // _TN)
    return pl.pallas_call(
        _argmin_body,
        grid=grid,
        in_specs=[
            pl.BlockSpec((_TM,), lambda r, c: (r,)),
            pl.BlockSpec((_TN,), lambda r, c: (c,)),
            pl.BlockSpec((_TM, _EMBED), lambda r, c: (r, 0)),
            pl.BlockSpec((_TN, _EMBED), lambda r, c: (c, 0)),
        ],
        out_specs=[
            pl.BlockSpec((_TM,), lambda r, c: (r,)),
            pl.BlockSpec((1, 1), lambda r, c: (0, 0),
                         memory_space=pltpu.SMEM),
        ],
        out_shape=[
            jax.ShapeDtypeStruct((_N,), jnp.int32),
            jax.ShapeDtypeStruct((1, 1), jnp.float32),
        ],
        scratch_shapes=[
            pltpu.VMEM((_TM,), jnp.float32),
            pltpu.VMEM((_TM,), jnp.int32),
            pltpu.VMEM((_TM,), jnp.float32),
            pltpu.VMEM((_TM,), jnp.int32),
            pltpu.VMEM((_TM,), jnp.float32),
            pltpu.VMEM((_TM,), jnp.int32),
            pltpu.SMEM((1,), jnp.float32),
        ],
        compiler_params=pltpu.CompilerParams(
            dimension_semantics=("arbitrary", "arbitrary")),
        interpret=interpret,
    )(zn, cn, z_flat, codebook)


_NW = 32
_CH = 128


def _gather_call(codebook, codes):
    b_per_w = _N // _NW
    nch = b_per_w // _CH
    idx2 = codes.reshape(_N // _CH, _CH)
    mesh = plsc.VectorSubcoreMesh(core_axis_name="c", subcore_axis_name="s")

    @functools.partial(
        pl.kernel,
        mesh=mesh,
        out_type=jax.ShapeDtypeStruct((_N, _EMBED), jnp.float32),
        scratch_types=[
            pltpu.VMEM((nch, _CH), jnp.int32),
            pltpu.VMEM((_CH, _EMBED), jnp.float32),
            pltpu.SemaphoreType.DMA,
        ],
    )
    def k(cb_hbm, idx_hbm, out_hbm, idx_v, rows_v, sem):
        wid = lax.axis_index("s") * 2 + lax.axis_index("c")
        pltpu.sync_copy(idx_hbm.at[pl.ds(wid * nch, nch)], idx_v)
        for ch in range(nch):
            pltpu.async_copy(cb_hbm.at[idx_v.at[ch]], rows_v, sem).wait()
            pltpu.sync_copy(
                rows_v, out_hbm.at[pl.ds(wid * b_per_w + ch * _CH, _CH)])

    return k(codebook, idx2)


def kernel(z, codebook):
    shape = z.shape
    z_flat = z.reshape(-1, _EMBED)
    zn = (z_flat ** 2).sum(axis=1)
    cn = (codebook ** 2).sum(axis=1)
    codes, loss = _argmin_call(zn, cn, z_flat, codebook)
    out = _gather_call(codebook, codes)
    return (out.reshape(shape), codes.reshape(shape[:-1]),
            loss.reshape(()))
